# baseline (device time: 92375 ns/iter reference)
import jax
import jax.numpy as jnp
from jax import lax
from jax.experimental import pallas as pl
from jax.experimental.pallas import tpu as pltpu

M = 8192
N_OUT = 1024
QROWS = M // 4
CH = 256
KQ = QROWS // CH
NCHUNKS = M // CH

DIAG_VIA_X = (0, 1)
DIAG_VIA_Z = (2, 3)
DIAG_VIA_Y = (4, 5, 6, 7)
KY = KQ + len(DIAG_VIA_Y)
KX = KQ + len(DIAG_VIA_X)
KZ = KQ + len(DIAG_VIA_Z)


def kernel(x):

    def body(
        x_ref,
        out_ref,
        rbuf,
        ysend,
        lstage_s,
        lstage_k,
        obuf,
        csem_s,
        csem_k,
        osem,
        ssem_y,
        rsem_y,
        ssem_x,
        rsem_x,
        ssem_z,
        rsem_z,
    ):
        my_x = lax.axis_index("x")
        my_y = lax.axis_index("y")
        my_z = lax.axis_index("z")
        yn = (my_x, 1 - my_y, my_z)
        xn = (1 - my_x, my_y, my_z)
        zn = (my_x, my_y, 1 - my_z)
        qme = 2 * my_z + my_x
        qxn = 2 * my_z + (1 - my_x)
        qzn = 2 * (1 - my_z) + my_x
        qdg = 2 * (1 - my_z) + (1 - my_x)
        send_col = (1 - my_y) * N_OUT
        keep_col = my_y * N_OUT

        barrier_sem = pltpu.get_barrier_semaphore()
        for nbr in (yn, xn, zn):
            pl.semaphore_signal(
                barrier_sem,
                inc=1,
                device_id=nbr,
                device_id_type=pl.DeviceIdType.MESH,
            )
        pl.semaphore_wait(barrier_sem, 3)

        def start_stage(q, c, col, stage, sem, slot, nch=1):
            cp = pltpu.make_async_copy(
                x_ref.at[0, pl.ds((q * KQ + c) * CH, nch * CH), pl.ds(col, N_OUT)],
                stage.at[slot, pl.ds(0, nch * CH)],
                sem.at[slot],
            )
            cp.start()
            return cp

        def swap_rdma(j, ssem, rsem, si, target):
            return pltpu.make_async_remote_copy(
                src_ref=rbuf.at[j],
                dst_ref=rbuf.at[j],
                send_sem=ssem.at[si],
                recv_sem=rsem.at[si],
                device_id=target,
                device_id_type=pl.DeviceIdType.MESH,
            )

        oc = [0]
        pend = [None] * 4

        def add_block(q, c0, nch, slot):
            s = oc[0] % 4
            if pend[s] is not None:
                pend[s].wait()
            j = q * KQ + c0
            for k in range(nch):
                obuf[s, pl.ds(k * CH, CH), :] = (
                    lstage_k[slot, pl.ds(k * CH, CH)].astype(jnp.bfloat16)
                    + rbuf[j + k]
                )
            cp = pltpu.make_async_copy(
                obuf.at[s, pl.ds(0, nch * CH)],
                out_ref.at[pl.ds(j * CH, nch * CH), :],
                osem.at[s],
            )
            cp.start()
            pend[s] = cp
            oc[0] += 1

        def run_phase(groups):
            n = len(groups)
            ks = [None] * n
            q0, c0, nch0, _ = groups[0]
            ks[0] = start_stage(q0, c0, keep_col, lstage_k, csem_k, 0, nch0)
            for i, (q, c0g, nch, handlers) in enumerate(groups):
                if i + 1 < n:
                    qn_, cn_, nchn, _ = groups[i + 1]
                    ks[i + 1] = start_stage(
                        qn_, cn_, keep_col, lstage_k, csem_k, (i + 1) % 2, nchn
                    )
                for h in handlers:
                    h()
                ks[i].wait()
                add_block(q, c0g, nch, i % 2)

        aseq = [(qme, c) for c in range(KQ)] + [(qdg, c) for c in DIAG_VIA_Y]
        agroups = []
        i = 0
        while i < len(aseq):
            q, c = aseq[i]
            if i + 1 < len(aseq) and aseq[i + 1] == (q, c + 1):
                agroups.append((q, c, 2))
                i += 2
            else:
                agroups.append((q, c, 1))
                i += 1
        rdy = [None] * KY
        cps = [None] * len(agroups)
        cps[0] = start_stage(
            agroups[0][0], agroups[0][1], send_col, lstage_s, csem_s, 0, agroups[0][2]
        )
        yi = 0
        for g, (q, c, nch) in enumerate(agroups):
            if g + 1 < len(agroups):
                qn_, cn_, nchn = agroups[g + 1]
                cps[g + 1] = start_stage(
                    qn_, cn_, send_col, lstage_s, csem_s, (g + 1) % 2, nchn
                )
            cps[g].wait()
            for k in range(nch):
                ysend[yi, :, :] = lstage_s[g % 2, pl.ds(k * CH, CH)].astype(
                    jnp.bfloat16
                )
                rd = pltpu.make_async_remote_copy(
                    src_ref=ysend.at[yi],
                    dst_ref=rbuf.at[(q + 0) * KQ + c + k],
                    send_sem=ssem_y.at[yi],
                    recv_sem=rsem_y.at[yi],
                    device_id=yn,
                    device_id_type=pl.DeviceIdType.MESH,
                )
                rd.start()
                rdy[yi] = rd
                yi += 1

        rdx_out = [None] * KX
        rdz_out = [None] * KZ

        def handle_b(c):
            def h():
                rdy[c].wait_recv()
                j = qme * KQ + c
                rdx_out[c] = swap_rdma(j, ssem_x, rsem_x, c, xn)
                rdx_out[c].start()
                rdz_out[c] = swap_rdma(j, ssem_z, rsem_z, c, zn)
                rdz_out[c].start()
            return h

        run_phase(
            [(qme, c, 2, [handle_b(c), handle_b(c + 1)]) for c in range(0, KQ, 2)]
        )

        def handle_cx(c):
            def h():
                j = qxn * KQ + c
                swap_rdma(j, ssem_x, rsem_x, c, xn).wait_recv()
                if c in DIAG_VIA_Z:
                    si = KQ + DIAG_VIA_Z.index(c)
                    rdz_out[si] = swap_rdma(j, ssem_z, rsem_z, si, zn)
                    rdz_out[si].start()
            return h

        def handle_cz(c):
            def h():
                j = qzn * KQ + c
                swap_rdma(j, ssem_z, rsem_z, c, zn).wait_recv()
                if c in DIAG_VIA_X:
                    si = KQ + DIAG_VIA_X.index(c)
                    rdx_out[si] = swap_rdma(j, ssem_x, rsem_x, si, xn)
                    rdx_out[si].start()
            return h

        cgroups = []
        for c in range(0, KQ, 2):
            cgroups.append((qxn, c, 2, [handle_cx(c), handle_cx(c + 1)]))
            cgroups.append((qzn, c, 2, [handle_cz(c), handle_cz(c + 1)]))
        run_phase(cgroups)

        def handle_d(c):
            def h():
                j = qdg * KQ + c
                if c in DIAG_VIA_Y:
                    rdy[KQ + DIAG_VIA_Y.index(c)].wait_recv()
                elif c in DIAG_VIA_Z:
                    swap_rdma(
                        j, ssem_z, rsem_z, KQ + DIAG_VIA_Z.index(c), zn
                    ).wait_recv()
                else:
                    swap_rdma(
                        j, ssem_x, rsem_x, KQ + DIAG_VIA_X.index(c), xn
                    ).wait_recv()
            return h

        dgroups = [
            (qdg, 4, 2, [handle_d(4), handle_d(5)]),
            (qdg, 6, 2, [handle_d(6), handle_d(7)]),
            (qdg, 2, 2, [handle_d(2), handle_d(3)]),
            (qdg, 0, 2, [handle_d(0), handle_d(1)]),
        ]
        run_phase(dgroups)

        for rd in rdy:
            rd.wait_send()
        for rd in rdx_out:
            rd.wait_send()
        for rd in rdz_out:
            rd.wait_send()
        for cp in pend:
            cp.wait()

    return pl.pallas_call(
        body,
        out_shape=jax.ShapeDtypeStruct((M, N_OUT), jnp.bfloat16),
        in_specs=[pl.BlockSpec(memory_space=pl.ANY)],
        out_specs=pl.BlockSpec(memory_space=pl.ANY),
        scratch_shapes=[
            pltpu.VMEM((NCHUNKS, CH, N_OUT), jnp.bfloat16),
            pltpu.VMEM((KY, CH, N_OUT), jnp.bfloat16),
            pltpu.VMEM((2, 2 * CH, N_OUT), jnp.float32),
            pltpu.VMEM((2, 2 * CH, N_OUT), jnp.float32),
            pltpu.VMEM((4, 2 * CH, N_OUT), jnp.bfloat16),
            pltpu.SemaphoreType.DMA((2,)),
            pltpu.SemaphoreType.DMA((2,)),
            pltpu.SemaphoreType.DMA((4,)),
            pltpu.SemaphoreType.DMA((KY,)),
            pltpu.SemaphoreType.DMA((KY,)),
            pltpu.SemaphoreType.DMA((KX,)),
            pltpu.SemaphoreType.DMA((KX,)),
            pltpu.SemaphoreType.DMA((KZ,)),
            pltpu.SemaphoreType.DMA((KZ,)),
        ],
        compiler_params=pltpu.CompilerParams(
            vmem_limit_bytes=60 * 1024 * 1024,
            collective_id=0,
        ),
    )(x)


# device time: 90972 ns/iter; 1.0154x vs baseline; 1.0154x over previous
import jax
import jax.numpy as jnp
from jax import lax
from jax.experimental import pallas as pl
from jax.experimental.pallas import tpu as pltpu

M = 8192
N_OUT = 1024
QROWS = M // 4
CH = 256
KQ = QROWS // CH
NCHUNKS = M // CH

DIAG_VIA_X = (0, 1, 2)
DIAG_VIA_Z = (3, 4)
DIAG_VIA_Y = (5, 6, 7)
KY = KQ + len(DIAG_VIA_Y)
KX = KQ + len(DIAG_VIA_X)
KZ = KQ + len(DIAG_VIA_Z)


def kernel(x):

    def body(
        x_ref,
        out_ref,
        rbuf,
        ysend,
        lstage_s,
        lstage_k,
        obuf,
        csem_s,
        csem_k,
        osem,
        ssem_y,
        rsem_y,
        ssem_x,
        rsem_x,
        ssem_z,
        rsem_z,
    ):
        my_x = lax.axis_index("x")
        my_y = lax.axis_index("y")
        my_z = lax.axis_index("z")
        yn = (my_x, 1 - my_y, my_z)
        xn = (1 - my_x, my_y, my_z)
        zn = (my_x, my_y, 1 - my_z)
        qme = 2 * my_z + my_x
        qxn = 2 * my_z + (1 - my_x)
        qzn = 2 * (1 - my_z) + my_x
        qdg = 2 * (1 - my_z) + (1 - my_x)
        send_col = (1 - my_y) * N_OUT
        keep_col = my_y * N_OUT

        barrier_sem = pltpu.get_barrier_semaphore()
        for nbr in (yn, xn, zn):
            pl.semaphore_signal(
                barrier_sem,
                inc=1,
                device_id=nbr,
                device_id_type=pl.DeviceIdType.MESH,
            )
        pl.semaphore_wait(barrier_sem, 3)

        def start_stage(q, c, col, stage, sem, slot):
            cp = pltpu.make_async_copy(
                x_ref.at[0, pl.ds((q * KQ + c) * CH, CH), pl.ds(col, N_OUT)],
                stage.at[slot],
                sem.at[slot],
            )
            cp.start()
            return cp

        def swap_rdma(j, ssem, rsem, si, target):
            return pltpu.make_async_remote_copy(
                src_ref=rbuf.at[j],
                dst_ref=rbuf.at[j],
                send_sem=ssem.at[si],
                recv_sem=rsem.at[si],
                device_id=target,
                device_id_type=pl.DeviceIdType.MESH,
            )

        oc = [0]
        pend = [None] * 4

        def add_chunk(j, slot):
            s = oc[0] % 4
            if pend[s] is not None:
                pend[s].wait()
            obuf[s, :, :] = lstage_k[slot].astype(jnp.bfloat16) + rbuf[j]
            cp = pltpu.make_async_copy(
                obuf.at[s], out_ref.at[pl.ds(j * CH, CH), :], osem.at[s]
            )
            cp.start()
            pend[s] = cp
            oc[0] += 1

        def run_phase(items):
            n = len(items)
            ks = [None] * n
            q0, c0, _ = items[0]
            ks[0] = start_stage(q0, c0, keep_col, lstage_k, csem_k, 0)
            for i, (q, c, handler) in enumerate(items):
                if i + 1 < n:
                    qn_, cn_, _ = items[i + 1]
                    ks[i + 1] = start_stage(
                        qn_, cn_, keep_col, lstage_k, csem_k, (i + 1) % 2
                    )
                handler()
                ks[i].wait()
                add_chunk(q * KQ + c, i % 2)

        aseq = [(qme, c) for c in range(KQ)] + [(qdg, c) for c in DIAG_VIA_Y]
        rdy = [None] * KY
        cps = [None] * KY
        cps[0] = start_stage(aseq[0][0], aseq[0][1], send_col, lstage_s, csem_s, 0)
        for i, (q, c) in enumerate(aseq):
            if i + 1 < KY:
                qn_, cn_ = aseq[i + 1]
                cps[i + 1] = start_stage(
                    qn_, cn_, send_col, lstage_s, csem_s, (i + 1) % 2
                )
            cps[i].wait()
            ysend[i, :, :] = lstage_s[i % 2].astype(jnp.bfloat16)
            rd = pltpu.make_async_remote_copy(
                src_ref=ysend.at[i],
                dst_ref=rbuf.at[q * KQ + c],
                send_sem=ssem_y.at[i],
                recv_sem=rsem_y.at[i],
                device_id=yn,
                device_id_type=pl.DeviceIdType.MESH,
            )
            rd.start()
            rdy[i] = rd

        rdx_out = [None] * KX
        rdz_out = [None] * KZ

        def handle_b(c):
            def h():
                rdy[c].wait_recv()
                j = qme * KQ + c
                rdx_out[c] = swap_rdma(j, ssem_x, rsem_x, c, xn)
                rdx_out[c].start()
                rdz_out[c] = swap_rdma(j, ssem_z, rsem_z, c, zn)
                rdz_out[c].start()
            return h

        run_phase([(qme, c, handle_b(c)) for c in range(KQ)])

        def handle_cx(c):
            def h():
                j = qxn * KQ + c
                swap_rdma(j, ssem_x, rsem_x, c, xn).wait_recv()
                if c in DIAG_VIA_Z:
                    si = KQ + DIAG_VIA_Z.index(c)
                    rdz_out[si] = swap_rdma(j, ssem_z, rsem_z, si, zn)
                    rdz_out[si].start()
            return h

        def handle_cz(c):
            def h():
                j = qzn * KQ + c
                swap_rdma(j, ssem_z, rsem_z, c, zn).wait_recv()
                if c in DIAG_VIA_X:
                    si = KQ + DIAG_VIA_X.index(c)
                    rdx_out[si] = swap_rdma(j, ssem_x, rsem_x, si, xn)
                    rdx_out[si].start()
            return h

        citems = []
        for c in range(KQ):
            citems.append((qxn, c, handle_cx(c)))
            citems.append((qzn, c, handle_cz(c)))
        run_phase(citems)

        def handle_d(c):
            def h():
                j = qdg * KQ + c
                if c in DIAG_VIA_Y:
                    rdy[KQ + DIAG_VIA_Y.index(c)].wait_recv()
                elif c in DIAG_VIA_Z:
                    swap_rdma(
                        j, ssem_z, rsem_z, KQ + DIAG_VIA_Z.index(c), zn
                    ).wait_recv()
                else:
                    swap_rdma(
                        j, ssem_x, rsem_x, KQ + DIAG_VIA_X.index(c), xn
                    ).wait_recv()
            return h

        dorder = list(DIAG_VIA_Y) + list(DIAG_VIA_Z) + list(DIAG_VIA_X)
        run_phase([(qdg, c, handle_d(c)) for c in dorder])

        for rd in rdy:
            rd.wait_send()
        for rd in rdx_out:
            rd.wait_send()
        for rd in rdz_out:
            rd.wait_send()
        for cp in pend:
            cp.wait()

    return pl.pallas_call(
        body,
        out_shape=jax.ShapeDtypeStruct((M, N_OUT), jnp.bfloat16),
        in_specs=[pl.BlockSpec(memory_space=pl.ANY)],
        out_specs=pl.BlockSpec(memory_space=pl.ANY),
        scratch_shapes=[
            pltpu.VMEM((NCHUNKS, CH, N_OUT), jnp.bfloat16),
            pltpu.VMEM((KY, CH, N_OUT), jnp.bfloat16),
            pltpu.VMEM((2, CH, N_OUT), jnp.float32),
            pltpu.VMEM((2, CH, N_OUT), jnp.float32),
            pltpu.VMEM((4, CH, N_OUT), jnp.bfloat16),
            pltpu.SemaphoreType.DMA((2,)),
            pltpu.SemaphoreType.DMA((2,)),
            pltpu.SemaphoreType.DMA((4,)),
            pltpu.SemaphoreType.DMA((KY,)),
            pltpu.SemaphoreType.DMA((KY,)),
            pltpu.SemaphoreType.DMA((KX,)),
            pltpu.SemaphoreType.DMA((KX,)),
            pltpu.SemaphoreType.DMA((KZ,)),
            pltpu.SemaphoreType.DMA((KZ,)),
        ],
        compiler_params=pltpu.CompilerParams(
            vmem_limit_bytes=60 * 1024 * 1024,
            collective_id=0,
        ),
    )(x)
